# Initial kernel scaffold; baseline (speedup 1.0000x reference)
import functools

import jax
import jax.numpy as jnp
from jax import lax
from jax.experimental import pallas as pl
from jax.experimental.pallas import tpu as pltpu
from jax.experimental.pallas import tpu_sc as plsc

NC = 2
NS = 16
LANES = 16
CHUNK = 128


def _degree_kernel(EP, NP):
    nw = NC * NS
    per_tile = EP // nw
    n_chunks = per_tile // CHUNK
    mesh = plsc.VectorSubcoreMesh(
        core_axis_name="c", subcore_axis_name="s", num_cores=NC, num_subcores=NS)

    @functools.partial(
        pl.kernel,
        out_type=(jax.ShapeDtypeStruct((nw, NP), jnp.float32),
                  jax.ShapeDtypeStruct((nw, NP), jnp.float32)),
        mesh=mesh,
        scratch_types=[
            pltpu.VMEM((NP,), jnp.float32),
            pltpu.VMEM((NP,), jnp.float32),
            pltpu.VMEM((CHUNK,), jnp.int32),
            pltpu.VMEM((CHUNK,), jnp.int32),
        ],
    )
    def deg_kernel(src_hbm, dst_hbm, outs_hbm, outd_hbm, hs, hd, sidx, didx):
        wid = lax.axis_index("s") * NC + lax.axis_index("c")
        zero16 = jnp.zeros((LANES,), jnp.float32)
        one16 = jnp.ones((LANES,), jnp.float32)

        def zero_body(i, carry):
            hs[pl.ds(i * LANES, LANES)] = zero16
            hd[pl.ds(i * LANES, LANES)] = zero16
            return carry

        lax.fori_loop(0, NP // LANES, zero_body, 0)

        base0 = wid * per_tile

        def chunk_body(c, carry):
            base = base0 + c * CHUNK
            pltpu.sync_copy(src_hbm.at[pl.ds(base, CHUNK)], sidx)
            pltpu.sync_copy(dst_hbm.at[pl.ds(base, CHUNK)], didx)

            def vec_body(j, carry2):
                iv_s = sidx[pl.ds(j * LANES, LANES)]
                iv_d = didx[pl.ds(j * LANES, LANES)]
                plsc.addupdate_scatter(hs, [iv_s], one16)
                plsc.addupdate_scatter(hd, [iv_d], one16)
                return carry2

            lax.fori_loop(0, CHUNK // LANES, vec_body, 0)
            return carry

        lax.fori_loop(0, n_chunks, chunk_body, 0)

        pltpu.sync_copy(hs, outs_hbm.at[wid])
        pltpu.sync_copy(hd, outd_hbm.at[wid])

    return deg_kernel


def _agg_kernel(EP, NP, F):
    per_core = EP // NC
    per_tile = per_core // NS
    n_chunks = per_tile // CHUNK
    rows_per_tile = NP // NS
    copies = rows_per_tile // CHUNK
    mesh = plsc.VectorSubcoreMesh(
        core_axis_name="c", subcore_axis_name="s", num_cores=NC, num_subcores=NS)

    @functools.partial(
        pl.kernel,
        out_type=jax.ShapeDtypeStruct((NC, NP, F), jnp.float32),
        mesh=mesh,
        scratch_types=[
            pltpu.VMEM_SHARED((NP, F), jnp.float32),
            pltpu.VMEM((CHUNK, F), jnp.float32),
            pltpu.VMEM((CHUNK,), jnp.int32),
            pltpu.VMEM((CHUNK,), jnp.int32),
            pltpu.SemaphoreType.DMA,
        ],
    )
    def agg(table_hbm, src_hbm, dst_hbm, out_hbm, acc_sh, rows, sidx, didx, sem):
        cid = lax.axis_index("c")
        sid = lax.axis_index("s")
        zero16 = jnp.zeros((LANES,), jnp.float32)

        def zrow(i, carry):
            def zcol(j, carry2):
                rows[i, pl.ds(j * LANES, LANES)] = zero16
                return carry2

            lax.fori_loop(0, F // LANES, zcol, 0)
            return carry

        lax.fori_loop(0, CHUNK, zrow, 0)

        r0 = sid * rows_per_tile
        for k in range(copies):
            pltpu.sync_copy(rows, acc_sh.at[pl.ds(r0 + k * CHUNK, CHUNK)])
        plsc.subcore_barrier()

        base0 = cid * per_core + sid * per_tile

        def chunk_body(c, carry):
            base = base0 + c * CHUNK
            pltpu.sync_copy(src_hbm.at[pl.ds(base, CHUNK)], sidx)
            pltpu.sync_copy(dst_hbm.at[pl.ds(base, CHUNK)], didx)
            pltpu.async_copy(table_hbm.at[sidx], rows, sem).wait()
            pltpu.sync_copy(rows, acc_sh.at[didx], add=True)
            return carry

        lax.fori_loop(0, n_chunks, chunk_body, 0)
        plsc.subcore_barrier()

        for k in range(copies):
            sl = pl.ds(r0 + k * CHUNK, CHUNK)
            pltpu.sync_copy(acc_sh.at[sl], out_hbm.at[cid, sl])

    return agg


def _norms_and_mm1(ps, pd, feat_p, W1, NP, RB=1024):
    nw, _ = ps.shape
    D, H1 = W1.shape

    def body(ps_ref, pd_ref, f_ref, w_ref, ns_ref, nd_ref, hw_ref):
        degs = jnp.sum(ps_ref[...], axis=0)
        degd = jnp.sum(pd_ref[...], axis=0)
        ns = jnp.where(degs > 0, lax.rsqrt(jnp.maximum(degs, 1.0)), 0.0)
        nd = jnp.where(degd > 0, lax.rsqrt(jnp.maximum(degd, 1.0)), 0.0)
        ns_ref[...] = ns[:, None]
        nd_ref[...] = nd[:, None]
        h0 = f_ref[...] * ns[:, None]
        hw_ref[...] = jnp.dot(h0, w_ref[...], preferred_element_type=jnp.float32)

    return pl.pallas_call(
        body,
        grid=(NP // RB,),
        in_specs=[
            pl.BlockSpec((nw, RB), lambda i: (0, i)),
            pl.BlockSpec((nw, RB), lambda i: (0, i)),
            pl.BlockSpec((RB, D), lambda i: (i, 0)),
            pl.BlockSpec((D, H1), lambda i: (0, 0)),
        ],
        out_specs=[
            pl.BlockSpec((RB, 1), lambda i: (i, 0)),
            pl.BlockSpec((RB, 1), lambda i: (i, 0)),
            pl.BlockSpec((RB, H1), lambda i: (i, 0)),
        ],
        out_shape=[
            jax.ShapeDtypeStruct((NP, 1), jnp.float32),
            jax.ShapeDtypeStruct((NP, 1), jnp.float32),
            jax.ShapeDtypeStruct((NP, H1), jnp.float32),
        ],
    )(ps, pd, feat_p, W1)


def _layer1_epilogue_mm2(agg1, norm_s, norm_d, b1, W2, NP, RB=1024):
    H1, H2 = W2.shape

    def body(p_ref, ns_ref, nd_ref, b_ref, w_ref, out_ref):
        agg = p_ref[0] + p_ref[1]
        h1 = jnp.maximum(agg * nd_ref[...] + b_ref[...], 0.0)
        out_ref[...] = jnp.dot(h1 * ns_ref[...], w_ref[...],
                               preferred_element_type=jnp.float32)

    return pl.pallas_call(
        body,
        grid=(NP // RB,),
        in_specs=[
            pl.BlockSpec((NC, RB, H1), lambda i: (0, i, 0)),
            pl.BlockSpec((RB, 1), lambda i: (i, 0)),
            pl.BlockSpec((RB, 1), lambda i: (i, 0)),
            pl.BlockSpec((1, H1), lambda i: (0, 0)),
            pl.BlockSpec((H1, H2), lambda i: (0, 0)),
        ],
        out_specs=pl.BlockSpec((RB, H2), lambda i: (i, 0)),
        out_shape=jax.ShapeDtypeStruct((NP, H2), jnp.float32),
    )(agg1, norm_s, norm_d, b1, W2)


def _form_z(agg2, norm_d, b2, NP, RB=1024):
    H2 = agg2.shape[-1]

    def body(p_ref, nd_ref, b_ref, out_ref):
        out_ref[...] = (p_ref[0] + p_ref[1]) * nd_ref[...] + b_ref[...]

    return pl.pallas_call(
        body,
        grid=(NP // RB,),
        in_specs=[
            pl.BlockSpec((NC, RB, H2), lambda i: (0, i, 0)),
            pl.BlockSpec((RB, 1), lambda i: (i, 0)),
            pl.BlockSpec((1, H2), lambda i: (0, 0)),
        ],
        out_specs=pl.BlockSpec((RB, H2), lambda i: (i, 0)),
        out_shape=jax.ShapeDtypeStruct((NP, H2), jnp.float32),
    )(agg2, norm_d, b2)


def _decoder(z, N, BM=1024, BN=1024):
    NP, H2 = z.shape
    gm = (N + BM - 1) // BM
    gn = (N + BN - 1) // BN

    def body(zr_ref, zc_ref, out_ref):
        logits = lax.dot_general(
            zr_ref[...], zc_ref[...],
            dimension_numbers=(((1,), (1,)), ((), ())),
            preferred_element_type=jnp.float32)
        out_ref[...] = jax.nn.sigmoid(logits)

    return pl.pallas_call(
        body,
        grid=(gm, gn),
        in_specs=[
            pl.BlockSpec((BM, H2), lambda i, j: (i, 0)),
            pl.BlockSpec((BN, H2), lambda i, j: (j, 0)),
        ],
        out_specs=pl.BlockSpec((BM, BN), lambda i, j: (i, j)),
        out_shape=jax.ShapeDtypeStruct((N, N), jnp.float32),
    )(z, z)


def kernel(features, edge_index, W1, b1, W2, b2):
    N, D = features.shape
    H1 = W1.shape[1]
    H2 = W2.shape[1]
    E = edge_index.shape[1]

    nw = NC * NS
    epg = nw * CHUNK
    EP = ((E + epg - 1) // epg) * epg
    npg = NS * CHUNK
    NP = ((N + 1 + npg - 1) // npg) * npg

    src = edge_index[0]
    dst = edge_index[1]
    pad = jnp.full((EP - E,), N, jnp.int32)
    src_p = jnp.concatenate([src, pad])
    dst_p = jnp.concatenate([dst, pad])
    feat_p = jnp.pad(features, ((0, NP - N), (0, 0)))

    ps, pd = _degree_kernel(EP, NP)(src_p, dst_p)
    norm_s, norm_d, hw1 = _norms_and_mm1(ps, pd, feat_p, W1, NP)
    agg1 = _agg_kernel(EP, NP, H1)(hw1, src_p, dst_p)
    hw2 = _layer1_epilogue_mm2(agg1, norm_s, norm_d, b1.reshape(1, H1), W2, NP)
    agg2 = _agg_kernel(EP, NP, H2)(hw2, src_p, dst_p)
    z = _form_z(agg2, norm_d, b2.reshape(1, H2), NP)
    return _decoder(z, N)


# speedup vs baseline: 2.6555x; 2.6555x over previous
import functools

import jax
import jax.numpy as jnp
from jax import lax
from jax.experimental import pallas as pl
from jax.experimental.pallas import tpu as pltpu
from jax.experimental.pallas import tpu_sc as plsc

NC = 2
NS = 16
LANES = 16
CHUNK = 128


DEGW = 16


def _agg_kernel(EP, NP, F):
    per_core = EP // NC
    per_tile = per_core // NS
    n_chunks = per_tile // CHUNK
    rows_per_tile = NP // NS
    copies = rows_per_tile // CHUNK
    mesh = plsc.VectorSubcoreMesh(
        core_axis_name="c", subcore_axis_name="s", num_cores=NC, num_subcores=NS)

    @functools.partial(
        pl.kernel,
        out_type=jax.ShapeDtypeStruct((NC, NP, F), jnp.float32),
        mesh=mesh,
        scratch_types=[
            pltpu.VMEM_SHARED((NP, F), jnp.float32),
            pltpu.VMEM((CHUNK, F), jnp.float32),
            pltpu.VMEM((CHUNK,), jnp.int32),
            pltpu.VMEM((CHUNK,), jnp.int32),
            pltpu.SemaphoreType.DMA,
        ],
        compiler_params=pltpu.CompilerParams(use_tc_tiling_on_sc=False),
    )
    def agg(table_hbm, src_hbm, dst_hbm, out_hbm, acc_sh, rows, sidx, didx, sem):
        cid = lax.axis_index("c")
        sid = lax.axis_index("s")
        zero16 = jnp.zeros((LANES,), jnp.float32)

        def zrow(i, carry):
            def zcol(j, carry2):
                rows[i, pl.ds(j * LANES, LANES)] = zero16
                return carry2

            lax.fori_loop(0, F // LANES, zcol, 0)
            return carry

        lax.fori_loop(0, CHUNK, zrow, 0)

        r0 = sid * rows_per_tile
        for k in range(copies):
            pltpu.sync_copy(rows, acc_sh.at[pl.ds(r0 + k * CHUNK, CHUNK)])
        plsc.subcore_barrier()

        base0 = cid * per_core + sid * per_tile

        def chunk_body(c, carry):
            base = base0 + c * CHUNK
            pltpu.sync_copy(src_hbm.at[pl.ds(base, CHUNK)], sidx)
            pltpu.sync_copy(dst_hbm.at[pl.ds(base, CHUNK)], didx)
            pltpu.async_copy(table_hbm.at[sidx], rows, sem).wait()
            pltpu.sync_copy(rows, acc_sh.at[didx], add=True)
            return carry

        lax.fori_loop(0, n_chunks, chunk_body, 0)
        plsc.subcore_barrier()

        for k in range(copies):
            sl = pl.ds(r0 + k * CHUNK, CHUNK)
            pltpu.sync_copy(acc_sh.at[sl], out_hbm.at[cid, sl])

    return agg


def _norms_and_mm1(ps, pd, feat_p, W1, NP, RB=1024):
    D, H1 = W1.shape

    def body(ps_ref, pd_ref, f_ref, w_ref, ns_ref, nd_ref, hw_ref):
        degs = ps_ref[0, :, 0] + ps_ref[1, :, 0]
        degd = pd_ref[0, :, 0] + pd_ref[1, :, 0]
        ns = jnp.where(degs > 0, lax.rsqrt(jnp.maximum(degs, 1.0)), 0.0)
        nd = jnp.where(degd > 0, lax.rsqrt(jnp.maximum(degd, 1.0)), 0.0)
        ns_ref[...] = ns[:, None]
        nd_ref[...] = nd[:, None]
        h0 = f_ref[...] * ns[:, None]
        hw_ref[...] = jnp.dot(h0, w_ref[...], preferred_element_type=jnp.float32)

    return pl.pallas_call(
        body,
        grid=(NP // RB,),
        in_specs=[
            pl.BlockSpec((NC, RB, DEGW), lambda i: (0, i, 0)),
            pl.BlockSpec((NC, RB, DEGW), lambda i: (0, i, 0)),
            pl.BlockSpec((RB, D), lambda i: (i, 0)),
            pl.BlockSpec((D, H1), lambda i: (0, 0)),
        ],
        out_specs=[
            pl.BlockSpec((RB, 1), lambda i: (i, 0)),
            pl.BlockSpec((RB, 1), lambda i: (i, 0)),
            pl.BlockSpec((RB, H1), lambda i: (i, 0)),
        ],
        out_shape=[
            jax.ShapeDtypeStruct((NP, 1), jnp.float32),
            jax.ShapeDtypeStruct((NP, 1), jnp.float32),
            jax.ShapeDtypeStruct((NP, H1), jnp.float32),
        ],
    )(ps, pd, feat_p, W1)


def _layer1_epilogue_mm2(agg1, norm_s, norm_d, b1, W2, NP, RB=1024):
    H1, H2 = W2.shape

    def body(p_ref, ns_ref, nd_ref, b_ref, w_ref, out_ref):
        agg = p_ref[0] + p_ref[1]
        h1 = jnp.maximum(agg * nd_ref[...] + b_ref[...], 0.0)
        out_ref[...] = jnp.dot(h1 * ns_ref[...], w_ref[...],
                               preferred_element_type=jnp.float32)

    return pl.pallas_call(
        body,
        grid=(NP // RB,),
        in_specs=[
            pl.BlockSpec((NC, RB, H1), lambda i: (0, i, 0)),
            pl.BlockSpec((RB, 1), lambda i: (i, 0)),
            pl.BlockSpec((RB, 1), lambda i: (i, 0)),
            pl.BlockSpec((1, H1), lambda i: (0, 0)),
            pl.BlockSpec((H1, H2), lambda i: (0, 0)),
        ],
        out_specs=pl.BlockSpec((RB, H2), lambda i: (i, 0)),
        out_shape=jax.ShapeDtypeStruct((NP, H2), jnp.float32),
    )(agg1, norm_s, norm_d, b1, W2)


def _form_z(agg2, norm_d, b2, NP, RB=1024):
    H2 = agg2.shape[-1]

    def body(p_ref, nd_ref, b_ref, out_ref):
        out_ref[...] = (p_ref[0] + p_ref[1]) * nd_ref[...] + b_ref[...]

    return pl.pallas_call(
        body,
        grid=(NP // RB,),
        in_specs=[
            pl.BlockSpec((NC, RB, H2), lambda i: (0, i, 0)),
            pl.BlockSpec((RB, 1), lambda i: (i, 0)),
            pl.BlockSpec((1, H2), lambda i: (0, 0)),
        ],
        out_specs=pl.BlockSpec((RB, H2), lambda i: (i, 0)),
        out_shape=jax.ShapeDtypeStruct((NP, H2), jnp.float32),
    )(agg2, norm_d, b2)


def _decoder(z, N, BM=1024, BN=1024):
    NP, H2 = z.shape
    gm = (N + BM - 1) // BM
    gn = (N + BN - 1) // BN

    def body(zr_ref, zc_ref, out_ref):
        logits = lax.dot_general(
            zr_ref[...], zc_ref[...],
            dimension_numbers=(((1,), (1,)), ((), ())),
            preferred_element_type=jnp.float32)
        out_ref[...] = jax.nn.sigmoid(logits)

    return pl.pallas_call(
        body,
        grid=(gm, gn),
        in_specs=[
            pl.BlockSpec((BM, H2), lambda i, j: (i, 0)),
            pl.BlockSpec((BN, H2), lambda i, j: (j, 0)),
        ],
        out_specs=pl.BlockSpec((BM, BN), lambda i, j: (i, j)),
        out_shape=jax.ShapeDtypeStruct((N, N), jnp.float32),
    )(z, z)


def kernel(features, edge_index, W1, b1, W2, b2):
    N, D = features.shape
    H1 = W1.shape[1]
    H2 = W2.shape[1]
    E = edge_index.shape[1]

    nw = NC * NS
    epg = nw * CHUNK
    EP = ((E + epg - 1) // epg) * epg
    npg = NS * CHUNK
    NP = ((N + 1 + npg - 1) // npg) * npg

    src = edge_index[0]
    dst = edge_index[1]
    pad = jnp.full((EP - E,), N, jnp.int32)
    src_p = jnp.concatenate([src, pad])
    dst_p = jnp.concatenate([dst, pad])
    feat_p = jnp.pad(features, ((0, NP - N), (0, 0)))

    ones_t = jnp.ones((NP, DEGW), jnp.float32)
    deg_agg = _agg_kernel(EP, NP, DEGW)
    ps = deg_agg(ones_t, dst_p, src_p)
    pd = deg_agg(ones_t, src_p, dst_p)
    norm_s, norm_d, hw1 = _norms_and_mm1(ps, pd, feat_p, W1, NP)
    agg1 = _agg_kernel(EP, NP, H1)(hw1, src_p, dst_p)
    hw2 = _layer1_epilogue_mm2(agg1, norm_s, norm_d, b1.reshape(1, H1), W2, NP)
    agg2 = _agg_kernel(EP, NP, H2)(hw2, src_p, dst_p)
    z = _form_z(agg2, norm_d, b2.reshape(1, H2), NP)
    return _decoder(z, N)


# speedup vs baseline: 3.6967x; 1.3921x over previous
import functools

import jax
import jax.numpy as jnp
from jax import lax
from jax.experimental import pallas as pl
from jax.experimental.pallas import tpu as pltpu
from jax.experimental.pallas import tpu_sc as plsc

NC = 2
NS = 16
LANES = 16
CHUNK = 128


DEGW = 16


NBUF = 4


def _agg_kernel(EP, NPT, NPA, F):
    nw = NC * NS
    per_tile = EP // nw
    n_chunks = per_tile // CHUNK
    rows_per_tile = NPA // NS
    copies = rows_per_tile // CHUNK
    assert n_chunks % NBUF == 0
    mesh = plsc.VectorSubcoreMesh(
        core_axis_name="c", subcore_axis_name="s", num_cores=NC, num_subcores=NS)

    @functools.partial(
        pl.kernel,
        out_type=jax.ShapeDtypeStruct((NC, NPA, F), jnp.float32),
        mesh=mesh,
        scratch_types=[
            pltpu.VMEM_SHARED((NPA, F), jnp.float32),
            pltpu.VMEM((NBUF, CHUNK, F), jnp.float32),
            pltpu.VMEM((n_chunks, CHUNK), jnp.int32),
            pltpu.VMEM((n_chunks, CHUNK), jnp.int32),
            pltpu.SemaphoreType.DMA((NBUF,)),
            pltpu.SemaphoreType.DMA((NBUF,)),
        ],
        compiler_params=pltpu.CompilerParams(use_tc_tiling_on_sc=False),
    )
    def agg(table_hbm, gidx_hbm, sidx_hbm, out_hbm,
            acc_sh, bufs, gidx, sidx, gsem, ssem):
        cid = lax.axis_index("c")
        sid = lax.axis_index("s")
        w = cid * NS + sid
        zero16 = jnp.zeros((LANES,), jnp.float32)

        def zrow(i, carry):
            def zcol(j, carry2):
                bufs[0, i, pl.ds(j * LANES, LANES)] = zero16
                return carry2

            lax.fori_loop(0, F // LANES, zcol, 0)
            return carry

        lax.fori_loop(0, CHUNK, zrow, 0)

        r0 = sid * rows_per_tile
        for k in range(copies):
            pltpu.sync_copy(bufs.at[0], acc_sh.at[pl.ds(r0 + k * CHUNK, CHUNK)])
        pltpu.sync_copy(gidx_hbm.at[w], gidx)
        pltpu.sync_copy(sidx_hbm.at[w], sidx)
        plsc.subcore_barrier()

        def gather_start(u, c):
            pltpu.make_async_copy(
                table_hbm.at[gidx.at[c]], bufs.at[u], gsem.at[u]).start()

        def gather_wait(u, c):
            pltpu.make_async_copy(
                table_hbm.at[gidx.at[c]], bufs.at[u], gsem.at[u]).wait()

        def scatter_start(u, c):
            pltpu.make_async_copy(
                bufs.at[u], acc_sh.at[sidx.at[c]], ssem.at[u]).start(add=True)

        def scatter_wait(u, c):
            pltpu.make_async_copy(
                bufs.at[u], acc_sh.at[sidx.at[c]], ssem.at[u]).wait()

        def pipe_body(i, carry):
            for u in range(NBUF):
                c = i * NBUF + u

                @pl.when(i > 0)
                def _():
                    scatter_wait(u, c - NBUF)

                gather_start(u, c)
            for u in range(NBUF):
                c = i * NBUF + u
                gather_wait(u, c)
                scatter_start(u, c)
            return carry

        lax.fori_loop(0, n_chunks // NBUF, pipe_body, 0)
        last = n_chunks - NBUF
        for u in range(NBUF):
            scatter_wait(u, last + u)
        plsc.subcore_barrier()

        for k in range(copies):
            sl = pl.ds(r0 + k * CHUNK, CHUNK)
            pltpu.sync_copy(acc_sh.at[sl], out_hbm.at[cid, sl])

    return agg


def _norms_and_mm1(pdeg, feat_p, W1, NP, RB=1024):
    D, H1 = W1.shape
    nblk = NP // RB

    def body(ps_ref, pd_ref, f_ref, w_ref, ns_ref, nd_ref, hw_ref):
        degs = ps_ref[0, :, 0] + ps_ref[1, :, 0]
        degd = pd_ref[0, :, 0] + pd_ref[1, :, 0]
        ns = jnp.where(degs > 0, lax.rsqrt(jnp.maximum(degs, 1.0)), 0.0)
        nd = jnp.where(degd > 0, lax.rsqrt(jnp.maximum(degd, 1.0)), 0.0)
        ns_ref[...] = ns[:, None]
        nd_ref[...] = nd[:, None]
        h0 = f_ref[...] * ns[:, None]
        hw_ref[...] = jnp.dot(h0, w_ref[...], preferred_element_type=jnp.float32)

    return pl.pallas_call(
        body,
        grid=(nblk,),
        in_specs=[
            pl.BlockSpec((NC, RB, DEGW), lambda i: (0, i, 0)),
            pl.BlockSpec((NC, RB, DEGW), lambda i: (0, i + nblk, 0)),
            pl.BlockSpec((RB, D), lambda i: (i, 0)),
            pl.BlockSpec((D, H1), lambda i: (0, 0)),
        ],
        out_specs=[
            pl.BlockSpec((RB, 1), lambda i: (i, 0)),
            pl.BlockSpec((RB, 1), lambda i: (i, 0)),
            pl.BlockSpec((RB, H1), lambda i: (i, 0)),
        ],
        out_shape=[
            jax.ShapeDtypeStruct((NP, 1), jnp.float32),
            jax.ShapeDtypeStruct((NP, 1), jnp.float32),
            jax.ShapeDtypeStruct((NP, H1), jnp.float32),
        ],
    )(pdeg, pdeg, feat_p, W1)


def _layer1_epilogue_mm2(agg1, norm_s, norm_d, b1, W2, NP, RB=1024):
    H1, H2 = W2.shape

    def body(p_ref, ns_ref, nd_ref, b_ref, w_ref, out_ref):
        agg = p_ref[0] + p_ref[1]
        h1 = jnp.maximum(agg * nd_ref[...] + b_ref[...], 0.0)
        out_ref[...] = jnp.dot(h1 * ns_ref[...], w_ref[...],
                               preferred_element_type=jnp.float32)

    return pl.pallas_call(
        body,
        grid=(NP // RB,),
        in_specs=[
            pl.BlockSpec((NC, RB, H1), lambda i: (0, i, 0)),
            pl.BlockSpec((RB, 1), lambda i: (i, 0)),
            pl.BlockSpec((RB, 1), lambda i: (i, 0)),
            pl.BlockSpec((1, H1), lambda i: (0, 0)),
            pl.BlockSpec((H1, H2), lambda i: (0, 0)),
        ],
        out_specs=pl.BlockSpec((RB, H2), lambda i: (i, 0)),
        out_shape=jax.ShapeDtypeStruct((NP, H2), jnp.float32),
    )(agg1, norm_s, norm_d, b1, W2)


def _form_z(agg2, norm_d, b2, NP, RB=1024):
    H2 = agg2.shape[-1]

    def body(p_ref, nd_ref, b_ref, out_ref):
        out_ref[...] = (p_ref[0] + p_ref[1]) * nd_ref[...] + b_ref[...]

    return pl.pallas_call(
        body,
        grid=(NP // RB,),
        in_specs=[
            pl.BlockSpec((NC, RB, H2), lambda i: (0, i, 0)),
            pl.BlockSpec((RB, 1), lambda i: (i, 0)),
            pl.BlockSpec((1, H2), lambda i: (0, 0)),
        ],
        out_specs=pl.BlockSpec((RB, H2), lambda i: (i, 0)),
        out_shape=jax.ShapeDtypeStruct((NP, H2), jnp.float32),
    )(agg2, norm_d, b2)


def _decoder(z, N, BM=1024, BN=1024):
    NP, H2 = z.shape
    gm = (N + BM - 1) // BM
    gn = (N + BN - 1) // BN

    def body(zr_ref, zc_ref, out_ref):
        logits = lax.dot_general(
            zr_ref[...], zc_ref[...],
            dimension_numbers=(((1,), (1,)), ((), ())),
            preferred_element_type=jnp.float32)
        out_ref[...] = jax.nn.sigmoid(logits)

    return pl.pallas_call(
        body,
        grid=(gm, gn),
        in_specs=[
            pl.BlockSpec((BM, H2), lambda i, j: (i, 0)),
            pl.BlockSpec((BN, H2), lambda i, j: (j, 0)),
        ],
        out_specs=pl.BlockSpec((BM, BN), lambda i, j: (i, j)),
        out_shape=jax.ShapeDtypeStruct((N, N), jnp.float32),
    )(z, z)


def kernel(features, edge_index, W1, b1, W2, b2):
    N, D = features.shape
    H1 = W1.shape[1]
    H2 = W2.shape[1]
    E = edge_index.shape[1]

    nw = NC * NS
    epg = nw * CHUNK
    EP = ((E + epg - 1) // epg) * epg
    npg = NS * CHUNK
    NP = ((N + 1 + npg - 1) // npg) * npg

    src = edge_index[0]
    dst = edge_index[1]
    pad = jnp.full((EP - E,), N, jnp.int32)
    src_p = jnp.concatenate([src, pad])
    dst_p = jnp.concatenate([dst, pad])
    feat_p = jnp.pad(features, ((0, NP - N), (0, 0)))

    n_chunks = EP // (nw * CHUNK)
    src_t = src_p.reshape(nw, n_chunks, CHUNK)
    dst_t = dst_p.reshape(nw, n_chunks, CHUNK)
    deg_scatter = jnp.concatenate([src_p, dst_p + NP]).reshape(nw, 2 * n_chunks, CHUNK)
    deg_gather = jnp.concatenate([src_p, src_p]).reshape(nw, 2 * n_chunks, CHUNK)

    ones_t = jnp.ones((NP, DEGW), jnp.float32)
    pdeg = _agg_kernel(2 * EP, NP, 2 * NP, DEGW)(ones_t, deg_gather, deg_scatter)
    norm_s, norm_d, hw1 = _norms_and_mm1(pdeg, feat_p, W1, NP)
    agg1 = _agg_kernel(EP, NP, NP, H1)(hw1, src_t, dst_t)
    hw2 = _layer1_epilogue_mm2(agg1, norm_s, norm_d, b1.reshape(1, H1), W2, NP)
    agg2 = _agg_kernel(EP, NP, NP, H2)(hw2, src_t, dst_t)
    z = _form_z(agg2, norm_d, b2.reshape(1, H2), NP)
    return _decoder(z, N)
